# Initial kernel scaffold; baseline (speedup 1.0000x reference)
import jax
import jax.numpy as jnp
from jax.experimental import pallas as pl


def kernel(x, q_w, q_b, kv_w, kv_b, out_w, out_b, lepe_w, lepe_b):
    raise NotImplementedError("write your pallas kernel here")


# speedup vs baseline: 3.0373x; 3.0373x over previous
import functools
import math

import jax
import jax.numpy as jnp
from jax.experimental import pallas as pl
from jax.experimental.pallas import tpu as pltpu

DIM = 96
NWIN = 14
R = 16
L = R * R
NREG = NWIN * NWIN
TOPK = 4
NH = 8
HD = DIM // NH
SEQ = 4
H = W = NWIN * R
BAND = NWIN

import numpy as _np
_FW = [float(_np.float32(math.exp(2.0 - SEQ - t))) for t in range(SEQ - 1)]


def _prep_kernel(x_ref, qw_ref, qb_ref, kvw_ref, kvb_ref,
                 q_ref, k_ref, v_ref, qm_ref, km_ref):
    x = x_ref[...]
    now = x[SEQ - 1].reshape(BAND * L, DIM)
    his = (x[0] * _FW[0] + x[1] * _FW[1] + x[2] * _FW[2]).reshape(BAND * L, DIM)
    q = jax.lax.dot_general(now, qw_ref[...], (((1,), (1,)), ((), ())),
                            preferred_element_type=jnp.float32) + qb_ref[...]
    kv = jax.lax.dot_general(his, kvw_ref[...], (((1,), (1,)), ((), ())),
                             preferred_element_type=jnp.float32) + kvb_ref[...]
    q3 = q.reshape(BAND, L, DIM)
    k3 = kv[:, :DIM].reshape(BAND, L, DIM)
    v3 = kv[:, DIM:].reshape(BAND, L, DIM)
    q_ref[...] = q3
    k_ref[...] = k3
    v_ref[...] = v3
    qm_ref[...] = jnp.mean(q3, axis=1).reshape(1, BAND, DIM)
    km_ref[...] = jnp.mean(k3, axis=1).reshape(1, BAND, DIM)


def _route_kernel(qm_ref, km_ref, idx_ref):
    a = jax.lax.dot_general(qm_ref[...], km_ref[...], (((1,), (1,)), ((), ())),
                            preferred_element_type=jnp.float32)
    iota = jax.lax.broadcasted_iota(jnp.int32, (NREG, NREG), 1)
    cols = []
    for _ in range(TOPK):
        m = jnp.max(a, axis=1, keepdims=True)
        sel = a >= m
        pick = jnp.min(jnp.where(sel, iota, NREG), axis=1, keepdims=True)
        cols.append(pick)
        a = jnp.where(iota == pick, -jnp.inf, a)
    idx_ref[...] = jnp.concatenate(cols, axis=1)


def _attn_kernel(idx_ref, q_ref, k0, k1, k2, k3, v0, v1, v2, v3, o_ref):
    del idx_ref
    scale = jnp.float32(DIM ** -0.5)
    q = q_ref[0] * scale
    kcat = jnp.concatenate([k0[0], k1[0], k2[0], k3[0]], axis=0)
    vcat = jnp.concatenate([v0[0], v1[0], v2[0], v3[0]], axis=0)
    outs = []
    for h in range(NH):
        sl = slice(h * HD, (h + 1) * HD)
        st = jax.lax.dot_general(kcat[:, sl], q[:, sl],
                                 (((1,), (1,)), ((), ())),
                                 preferred_element_type=jnp.float32)
        mx = jnp.max(st, axis=0, keepdims=True)
        e = jnp.exp(st - mx)
        s = jnp.sum(e, axis=0, keepdims=True)
        ot = jax.lax.dot_general(vcat[:, sl], e, (((0,), (0,)), ((), ())),
                                 preferred_element_type=jnp.float32)
        outs.append(ot / s)
    o_ref[0] = jnp.concatenate(outs, axis=0)


def _final_kernel(at_ref, vu_ref, vc_ref, vd_ref, lw_ref, lb_ref,
                  ow_ref, ob_ref, o_ref):
    i = pl.program_id(0)
    vc = vc_ref[...]
    up = jnp.where(i > 0, vu_ref[:, R - 1, :], 0.0)
    dn = jnp.where(i < NWIN - 1, vd_ref[:, 0, :], 0.0)
    rows = jnp.concatenate([up[:, None, :], vc, dn[:, None, :]], axis=1)
    pad = jnp.pad(rows, ((0, 0), (0, 0), (1, 1)))
    lw = lw_ref[...]
    acc = lb_ref[...].reshape(DIM, 1, 1)
    acc = acc + sum(
        lw[:, 3 * dy + dx].reshape(DIM, 1, 1) * pad[:, dy:dy + R, dx:dx + W]
        for dy in range(3) for dx in range(3))
    y = (at_ref[...] + acc).reshape(DIM, R * W)
    out = jax.lax.dot_general(ow_ref[...], y, (((1,), (0,)), ((), ())),
                              preferred_element_type=jnp.float32)
    out = out + ob_ref[...].reshape(DIM, 1)
    o_ref[...] = out.reshape(DIM, R, W)


def kernel(x, q_w, q_b, kv_w, kv_b, out_w, out_b, lepe_w, lepe_b):
    f32 = jnp.float32
    x_reg = x.reshape(SEQ, DIM, NWIN, R, NWIN, R)
    x_reg = x_reg.transpose(0, 2, 4, 3, 5, 1).reshape(SEQ, NREG, L, DIM)

    q_reg, k_reg, v_reg, q_mean, k_mean = pl.pallas_call(
        _prep_kernel,
        grid=(NWIN,),
        in_specs=[
            pl.BlockSpec((SEQ, BAND, L, DIM), lambda i: (0, i, 0, 0)),
            pl.BlockSpec((DIM, DIM), lambda i: (0, 0)),
            pl.BlockSpec((DIM,), lambda i: (0,)),
            pl.BlockSpec((2 * DIM, DIM), lambda i: (0, 0)),
            pl.BlockSpec((2 * DIM,), lambda i: (0,)),
        ],
        out_specs=[
            pl.BlockSpec((BAND, L, DIM), lambda i: (i, 0, 0)),
            pl.BlockSpec((BAND, L, DIM), lambda i: (i, 0, 0)),
            pl.BlockSpec((BAND, L, DIM), lambda i: (i, 0, 0)),
            pl.BlockSpec((1, BAND, DIM), lambda i: (i, 0, 0)),
            pl.BlockSpec((1, BAND, DIM), lambda i: (i, 0, 0)),
        ],
        out_shape=[
            jax.ShapeDtypeStruct((NREG, L, DIM), f32),
            jax.ShapeDtypeStruct((NREG, L, DIM), f32),
            jax.ShapeDtypeStruct((NREG, L, DIM), f32),
            jax.ShapeDtypeStruct((NWIN, BAND, DIM), f32),
            jax.ShapeDtypeStruct((NWIN, BAND, DIM), f32),
        ],
    )(x_reg, q_w, q_b, kv_w, kv_b)

    idx = pl.pallas_call(
        _route_kernel,
        out_shape=jax.ShapeDtypeStruct((NREG, TOPK), jnp.int32),
    )(q_mean.reshape(NREG, DIM), k_mean.reshape(NREG, DIM))

    def _kv_spec(j):
        return pl.BlockSpec((1, L, DIM),
                            lambda n, idx_ref, j=j: (idx_ref[n, j], 0, 0))

    attn = pl.pallas_call(
        _attn_kernel,
        grid_spec=pltpu.PrefetchScalarGridSpec(
            num_scalar_prefetch=1,
            grid=(NREG,),
            in_specs=[pl.BlockSpec((1, L, DIM), lambda n, idx_ref: (n, 0, 0))]
                     + [_kv_spec(j) for j in range(TOPK)] * 2,
            out_specs=pl.BlockSpec((1, DIM, L), lambda n, idx_ref: (n, 0, 0)),
        ),
        out_shape=jax.ShapeDtypeStruct((NREG, DIM, L), f32),
    )(idx, q_reg, k_reg, k_reg, k_reg, k_reg, v_reg, v_reg, v_reg, v_reg)

    attn_plain = attn.reshape(NWIN, NWIN, DIM, R, R)
    attn_plain = attn_plain.transpose(2, 0, 3, 1, 4).reshape(DIM, H, W)
    v_plain = v_reg.reshape(NWIN, NWIN, R, R, DIM)
    v_plain = v_plain.transpose(4, 0, 2, 1, 3).reshape(DIM, H, W)
    lw = lepe_w.reshape(DIM, 9)

    def row(d):
        return lambda i: (0, jnp.clip(i + d, 0, NWIN - 1), 0)

    out = pl.pallas_call(
        _final_kernel,
        grid=(NWIN,),
        in_specs=[
            pl.BlockSpec((DIM, R, W), lambda i: (0, i, 0)),
            pl.BlockSpec((DIM, R, W), row(-1)),
            pl.BlockSpec((DIM, R, W), row(0)),
            pl.BlockSpec((DIM, R, W), row(1)),
            pl.BlockSpec((DIM, 9), lambda i: (0, 0)),
            pl.BlockSpec((DIM,), lambda i: (0,)),
            pl.BlockSpec((DIM, DIM), lambda i: (0, 0)),
            pl.BlockSpec((DIM,), lambda i: (0,)),
        ],
        out_specs=pl.BlockSpec((DIM, R, W), lambda i: (0, i, 0)),
        out_shape=jax.ShapeDtypeStruct((DIM, H, W), f32),
    )(attn_plain, v_plain, v_plain, v_plain, lw, lepe_b, out_w, out_b)

    return out.reshape(1, DIM, H, W)


# speedup vs baseline: 4.5761x; 1.5066x over previous
import functools
import math

import jax
import jax.numpy as jnp
import numpy as _np
from jax.experimental import pallas as pl
from jax.experimental.pallas import tpu as pltpu

DIM = 96
NWIN = 14
R = 16
L = R * R
NREG = NWIN * NWIN
TOPK = 4
NH = 8
HD = DIM // NH
SEQ = 4
H = W = NWIN * R

_FW = [float(_np.float32(math.exp(2.0 - SEQ - t))) for t in range(SEQ - 1)]


def _prep_kernel(x_ref, qw_ref, qb_ref, kvw_ref, kvb_ref,
                 q_ref, k_ref, v_ref, qm_ref, km_ref):
    x4 = x_ref[...]
    his3 = x4[0] * _FW[0] + x4[1] * _FW[1] + x4[2] * _FW[2]
    both = jnp.concatenate([x4[SEQ - 1], his3], axis=0)
    xt = jnp.transpose(both, (1, 2, 0)).reshape(R * W, 2 * DIM)
    now = xt[:, :DIM]
    his = xt[:, DIM:]
    q = jax.lax.dot_general(now, qw_ref[...], (((1,), (1,)), ((), ())),
                            preferred_element_type=jnp.float32) + qb_ref[...]
    kv = jax.lax.dot_general(his, kvw_ref[...], (((1,), (1,)), ((), ())),
                             preferred_element_type=jnp.float32) + kvb_ref[...]
    k = kv[:, :DIM]
    v = kv[:, DIM:]
    q4 = q.reshape(R, NWIN, R, DIM)
    k4 = k.reshape(R, NWIN, R, DIM)
    qm_ref[...] = jnp.mean(q4, axis=(0, 2)).reshape(1, NWIN, DIM)
    km_ref[...] = jnp.mean(k4, axis=(0, 2)).reshape(1, NWIN, DIM)
    q_ref[...] = q.astype(jnp.bfloat16).reshape(R, W, DIM)
    k_ref[...] = k.astype(jnp.bfloat16).reshape(R, W, DIM)
    v_ref[...] = v.astype(jnp.bfloat16).reshape(R, W, DIM)


def _route_kernel(qm_ref, km_ref, idx_ref):
    a = jax.lax.dot_general(qm_ref[...], km_ref[...], (((1,), (1,)), ((), ())),
                            preferred_element_type=jnp.float32)
    iota = jax.lax.broadcasted_iota(jnp.int32, (NREG, NREG), 1)
    cols = []
    for _ in range(TOPK):
        m = jnp.max(a, axis=1, keepdims=True)
        sel = a >= m
        pick = jnp.min(jnp.where(sel, iota, NREG), axis=1, keepdims=True)
        cols.append(pick)
        a = jnp.where(iota == pick, -jnp.inf, a)
    idx_ref[...] = jnp.concatenate(cols, axis=1)


def _attn_kernel(idx_ref, q_ref, k0, k1, k2, k3, v0, v1, v2, v3, o_ref):
    del idx_ref
    scale = jnp.bfloat16(DIM ** -0.5)
    q = q_ref[...].reshape(L, DIM) * scale
    ks = [r[...].reshape(L, DIM) for r in (k0, k1, k2, k3)]
    vs = [r[...].reshape(L, DIM) for r in (v0, v1, v2, v3)]
    outs = []
    for h in range(NH):
        sl = slice(h * HD, (h + 1) * HD)
        qh = q[:, sl]
        sts = [jax.lax.dot_general(kj[:, sl], qh, (((1,), (1,)), ((), ())),
                                   preferred_element_type=jnp.float32)
               for kj in ks]
        mx = functools.reduce(jnp.maximum,
                              [jnp.max(s, axis=0, keepdims=True) for s in sts])
        es = [jnp.exp(s - mx) for s in sts]
        ssum = sum(jnp.sum(e, axis=0, keepdims=True) for e in es)
        ot = sum(jax.lax.dot_general(vj[:, sl].astype(jnp.float32), e,
                                     (((0,), (0,)), ((), ())),
                                     preferred_element_type=jnp.float32)
                 for vj, e in zip(vs, es))
        outs.append(ot / ssum)
    o = jnp.concatenate(outs, axis=0)
    o_ref[...] = jnp.transpose(o, (1, 0)).reshape(R, R, DIM)


def _final_kernel(at_ref, vu_ref, vc_ref, vd_ref, lw_ref, lb_ref,
                  ow_ref, ob_ref, o_ref):
    i = pl.program_id(0)
    vc = vc_ref[...].astype(jnp.float32)
    up = jnp.where(i > 0, vu_ref[R - 1, :, :].astype(jnp.float32), 0.0)
    dn = jnp.where(i < NWIN - 1, vd_ref[0, :, :].astype(jnp.float32), 0.0)
    rows = jnp.concatenate([up[None], vc, dn[None]], axis=0)
    pad = jnp.pad(rows, ((0, 0), (1, 1), (0, 0)))
    lw = lw_ref[...]
    acc = lb_ref[...].reshape(1, 1, DIM)
    acc = acc + sum(
        lw[3 * dy + dx].reshape(1, 1, DIM) * pad[dy:dy + R, dx:dx + W, :]
        for dy in range(3) for dx in range(3))
    y = (at_ref[...] + acc).reshape(R * W, DIM)
    out = jax.lax.dot_general(y, ow_ref[...], (((1,), (1,)), ((), ())),
                              preferred_element_type=jnp.float32)
    out = out + ob_ref[...]
    o_ref[...] = jnp.transpose(out, (1, 0)).reshape(DIM, R, W)


def kernel(x, q_w, q_b, kv_w, kv_b, out_w, out_b, lepe_w, lepe_b):
    f32 = jnp.float32
    bf16 = jnp.bfloat16

    q_full, k_full, v_full, q_mean, k_mean = pl.pallas_call(
        _prep_kernel,
        grid=(NWIN,),
        in_specs=[
            pl.BlockSpec((SEQ, DIM, R, W), lambda i: (0, 0, i, 0)),
            pl.BlockSpec((DIM, DIM), lambda i: (0, 0)),
            pl.BlockSpec((DIM,), lambda i: (0,)),
            pl.BlockSpec((2 * DIM, DIM), lambda i: (0, 0)),
            pl.BlockSpec((2 * DIM,), lambda i: (0,)),
        ],
        out_specs=[
            pl.BlockSpec((R, W, DIM), lambda i: (i, 0, 0)),
            pl.BlockSpec((R, W, DIM), lambda i: (i, 0, 0)),
            pl.BlockSpec((R, W, DIM), lambda i: (i, 0, 0)),
            pl.BlockSpec((1, NWIN, DIM), lambda i: (i, 0, 0)),
            pl.BlockSpec((1, NWIN, DIM), lambda i: (i, 0, 0)),
        ],
        out_shape=[
            jax.ShapeDtypeStruct((H, W, DIM), bf16),
            jax.ShapeDtypeStruct((H, W, DIM), bf16),
            jax.ShapeDtypeStruct((H, W, DIM), bf16),
            jax.ShapeDtypeStruct((NWIN, NWIN, DIM), f32),
            jax.ShapeDtypeStruct((NWIN, NWIN, DIM), f32),
        ],
    )(x.reshape(SEQ, DIM, H, W), q_w, q_b, kv_w, kv_b)

    idx = pl.pallas_call(
        _route_kernel,
        out_shape=jax.ShapeDtypeStruct((NREG, TOPK), jnp.int32),
    )(q_mean.reshape(NREG, DIM), k_mean.reshape(NREG, DIM))

    def _kv_spec(j):
        return pl.BlockSpec(
            (R, R, DIM),
            lambda n, idx_ref, j=j: (idx_ref[n, j] // NWIN,
                                     idx_ref[n, j] % NWIN, 0))

    attn = pl.pallas_call(
        _attn_kernel,
        grid_spec=pltpu.PrefetchScalarGridSpec(
            num_scalar_prefetch=1,
            grid=(NREG,),
            in_specs=[pl.BlockSpec((R, R, DIM),
                                   lambda n, idx_ref: (n // NWIN, n % NWIN, 0))]
                     + [_kv_spec(j) for j in range(TOPK)] * 2,
            out_specs=pl.BlockSpec((R, R, DIM),
                                   lambda n, idx_ref: (n // NWIN, n % NWIN, 0)),
        ),
        out_shape=jax.ShapeDtypeStruct((H, W, DIM), f32),
    )(idx, q_full, k_full, k_full, k_full, k_full,
      v_full, v_full, v_full, v_full)

    def row(d):
        return lambda i: (jnp.clip(i + d, 0, NWIN - 1), 0, 0)

    out = pl.pallas_call(
        _final_kernel,
        grid=(NWIN,),
        in_specs=[
            pl.BlockSpec((R, W, DIM), lambda i: (i, 0, 0)),
            pl.BlockSpec((R, W, DIM), row(-1)),
            pl.BlockSpec((R, W, DIM), row(0)),
            pl.BlockSpec((R, W, DIM), row(1)),
            pl.BlockSpec((9, DIM), lambda i: (0, 0)),
            pl.BlockSpec((DIM,), lambda i: (0,)),
            pl.BlockSpec((DIM, DIM), lambda i: (0, 0)),
            pl.BlockSpec((DIM,), lambda i: (0,)),
        ],
        out_specs=pl.BlockSpec((DIM, R, W), lambda i: (0, i, 0)),
        out_shape=jax.ShapeDtypeStruct((DIM, H, W), f32),
    )(attn, v_full, v_full, v_full,
      lepe_w.reshape(DIM, 9).T, lepe_b, out_w, out_b)

    return out.reshape(1, DIM, H, W)
